# Initial kernel scaffold; baseline (speedup 1.0000x reference)
import functools

import jax
import jax.numpy as jnp
from jax import lax
from jax.experimental import pallas as pl
from jax.experimental.pallas import tpu as pltpu
from jax.experimental.pallas import tpu_sc as plsc

N = 10000
E = 320000
DIN = 128
DH = 256
DOUT = 128

NP = 10240
EP = 327680
EPR = EP // 128
RPT = NP // 16
EB = 8
BR = 1024

_MESH = plsc.VectorSubcoreMesh(core_axis_name="c", subcore_axis_name="s")


def _rsqrt_newton(d):
    i = plsc.bitcast(d, jnp.int32)
    i = jnp.int32(0x5F3759DF) - (i >> 1)
    y = plsc.bitcast(i, jnp.float32)
    for _ in range(3):
        y = y * (1.5 - 0.5 * d * y * y)
    return y


def _deg_body(dst_hbm, ones_hbm, zeros_hbm, dis_out, deg_sh, dstb, onesb,
              degv, disv, sem):
    c = lax.axis_index("c")
    s = lax.axis_index("s")

    @pl.when(c == 0)
    def _():
        rb = s * RPT
        pltpu.sync_copy(zeros_hbm.at[pl.ds(rb, RPT)], deg_sh.at[pl.ds(rb, RPT)])
        pltpu.sync_copy(ones_hbm, onesb)
        plsc.subcore_barrier()

        ept_rows = EPR // 16
        row0 = s * ept_rows

        def blk(b, _):
            pltpu.sync_copy(dst_hbm.at[pl.ds(row0 + b * EB, EB)], dstb)

            def j_body(j, _):
                pltpu.sync_copy(onesb, deg_sh.at[dstb.at[j]], add=True)
                return 0

            return lax.fori_loop(0, EB, j_body, 0)

        lax.fori_loop(0, ept_rows // EB, blk, 0)
        plsc.subcore_barrier()

        pltpu.sync_copy(deg_sh.at[pl.ds(rb, RPT)], degv)

        def e_body(q, _):
            ridx = lax.iota(jnp.int32, 16) + q * 16
            cidx = jnp.zeros((16,), jnp.int32)
            g = plsc.load_gather(degv, [ridx, cidx])
            disv[pl.ds(q * 16, 16)] = _rsqrt_newton(g + 1.0)
            return 0

        lax.fori_loop(0, RPT // 16, e_body, 0)
        pltpu.sync_copy(disv, dis_out.at[pl.ds(rb, RPT)])


def _deg_kernel(dst2d, ones8, zeros8):
    return pl.kernel(
        _deg_body,
        out_type=jax.ShapeDtypeStruct((NP,), jnp.float32),
        mesh=_MESH,
        scratch_types=[
            pltpu.VMEM_SHARED((NP, 8), jnp.float32),
            pltpu.VMEM((EB, 128), jnp.int32),
            pltpu.VMEM((128, 8), jnp.float32),
            pltpu.VMEM((RPT, 8), jnp.float32),
            pltpu.VMEM((RPT,), jnp.float32),
            pltpu.SemaphoreType.DMA,
        ],
    )(dst2d, ones8, zeros8)


def _make_agg_body(feature_split):
    def body(table, src_hbm, dst_hbm, zeros_hbm, out, agg_sh, srcb, dstb,
             idxg, rows, gsem):
        c = lax.axis_index("c")
        s = lax.axis_index("s")
        rb = s * RPT
        pltpu.sync_copy(zeros_hbm.at[pl.ds(rb, RPT)], agg_sh.at[pl.ds(rb, RPT)])
        plsc.subcore_barrier()

        if feature_split:
            tile_rows = EPR // 16
            row0 = s * tile_rows
        else:
            tile_rows = EPR // 32
            row0 = (s * 2 + c) * tile_rows

        def blk(b, _):
            e0 = (row0 + b * EB) * 128
            pltpu.sync_copy(src_hbm.at[pl.ds(e0, EB * 128)], srcb)
            pltpu.sync_copy(dst_hbm.at[pl.ds(row0 + b * EB, EB)], dstb)

            def j_body(j, _):
                if feature_split:
                    def t_body(k, _):
                        v = srcb[pl.ds(j * 128 + k * 16, 16)]
                        idxg[pl.ds(k * 16, 16)] = v * 2 + c
                        return 0
                    lax.fori_loop(0, 8, t_body, 0)
                    gidx = idxg
                else:
                    gidx = srcb.at[pl.ds(j * 128, 128)]
                pltpu.async_copy(table.at[gidx], rows, gsem).wait()
                pltpu.sync_copy(rows, agg_sh.at[dstb.at[j]], add=True)
                return 0

            return lax.fori_loop(0, EB, j_body, 0)

        lax.fori_loop(0, tile_rows // EB, blk, 0)
        plsc.subcore_barrier()
        pltpu.sync_copy(agg_sh.at[pl.ds(rb, RPT)], out.at[c, pl.ds(rb, RPT)])

    return body


def _agg_kernel(table, src1d, dst2d, zeros128, feature_split):
    return pl.kernel(
        _make_agg_body(feature_split),
        out_type=jax.ShapeDtypeStruct((2, NP, 128), jnp.float32),
        mesh=_MESH,
        scratch_types=[
            pltpu.VMEM_SHARED((NP, 128), jnp.float32),
            pltpu.VMEM((EB * 128,), jnp.int32),
            pltpu.VMEM((EB, 128), jnp.int32),
            pltpu.VMEM((128,), jnp.int32),
            pltpu.VMEM((128, 128), jnp.float32),
            pltpu.SemaphoreType.DMA,
        ],
    )(table, src1d, dst2d, zeros128)


def _tc_matmul1(xp, W1, dis2):
    def body(x_ref, w_ref, d_ref, o_ref):
        xw = jnp.dot(x_ref[...], w_ref[...], preferred_element_type=jnp.float32)
        o_ref[...] = xw * d_ref[...]

    return pl.pallas_call(
        body,
        grid=(NP // BR,),
        in_specs=[
            pl.BlockSpec((BR, DIN), lambda i: (i, 0)),
            pl.BlockSpec((DIN, DH), lambda i: (0, 0)),
            pl.BlockSpec((BR, 1), lambda i: (i, 0)),
        ],
        out_specs=pl.BlockSpec((BR, DH), lambda i: (i, 0)),
        out_shape=jax.ShapeDtypeStruct((NP, DH), jnp.float32),
    )(xp, W1, dis2)


def _tc_mid(agg1, y1, dis2, b1, gam, bet, mu, var, W2):
    def body(a_ref, y_ref, d_ref, b_ref, g_ref, be_ref, m_ref, v_ref, w_ref,
             o_ref):
        a = jnp.concatenate([a_ref[0], a_ref[1]], axis=1)
        d = d_ref[...]
        g = d * (a + y_ref[...]) + b_ref[...]
        scale = g_ref[...] * lax.rsqrt(v_ref[...] + 1e-5)
        h = jnp.maximum(g * scale + (be_ref[...] - m_ref[...] * scale), 0.0)
        o_ref[...] = jnp.dot(h, w_ref[...],
                             preferred_element_type=jnp.float32) * d

    return pl.pallas_call(
        body,
        grid=(NP // BR,),
        in_specs=[
            pl.BlockSpec((2, BR, 128), lambda i: (0, i, 0)),
            pl.BlockSpec((BR, DH), lambda i: (i, 0)),
            pl.BlockSpec((BR, 1), lambda i: (i, 0)),
            pl.BlockSpec((1, DH), lambda i: (0, 0)),
            pl.BlockSpec((1, DH), lambda i: (0, 0)),
            pl.BlockSpec((1, DH), lambda i: (0, 0)),
            pl.BlockSpec((1, DH), lambda i: (0, 0)),
            pl.BlockSpec((1, DH), lambda i: (0, 0)),
            pl.BlockSpec((DH, DOUT), lambda i: (0, 0)),
        ],
        out_specs=pl.BlockSpec((BR, DOUT), lambda i: (i, 0)),
        out_shape=jax.ShapeDtypeStruct((NP, DOUT), jnp.float32),
    )(agg1, y1, dis2, b1, gam, bet, mu, var, W2)


def _tc_final(agg2, y2, dis2, b2):
    def body(a_ref, y_ref, d_ref, b_ref, o_ref):
        o_ref[...] = d_ref[...] * (a_ref[0] + a_ref[1] + y_ref[...]) + b_ref[...]

    return pl.pallas_call(
        body,
        grid=(NP // BR,),
        in_specs=[
            pl.BlockSpec((2, BR, DOUT), lambda i: (0, i, 0)),
            pl.BlockSpec((BR, DOUT), lambda i: (i, 0)),
            pl.BlockSpec((BR, 1), lambda i: (i, 0)),
            pl.BlockSpec((1, DOUT), lambda i: (0, 0)),
        ],
        out_specs=pl.BlockSpec((BR, DOUT), lambda i: (i, 0)),
        out_shape=jax.ShapeDtypeStruct((NP, DOUT), jnp.float32),
    )(agg2, y2, dis2, b2)


@jax.jit
def kernel(node_feat, edge_index, W1, b1, W2, b2, bn_gamma, bn_beta, bn_mean,
           bn_var):
    f32 = jnp.float32
    xp = jnp.zeros((NP, DIN), f32).at[:N].set(node_feat)
    pad = jnp.full((EP - E,), N, jnp.int32)
    src = jnp.concatenate([edge_index[0], pad])
    dst = jnp.concatenate([edge_index[1], pad])
    dst2d = dst.reshape(EPR, 128)

    ones8 = jnp.ones((128, 8), f32)
    zeros8 = jnp.zeros((NP, 8), f32)
    zeros128 = jnp.zeros((NP, 128), f32)

    dis = _deg_kernel(dst2d, ones8, zeros8)
    dis2 = dis.reshape(NP, 1)

    y1 = _tc_matmul1(xp, W1, dis2)
    agg1 = _agg_kernel(y1.reshape(2 * NP, 128), src, dst2d, zeros128,
                       feature_split=True)
    y2 = _tc_mid(agg1, y1, dis2, b1.reshape(1, DH), bn_gamma.reshape(1, DH),
                 bn_beta.reshape(1, DH), bn_mean.reshape(1, DH),
                 bn_var.reshape(1, DH), W2)
    agg2 = _agg_kernel(y2, src, dst2d, zeros128, feature_split=False)
    out = _tc_final(agg2, y2, dis2, b2.reshape(1, DOUT))
    return out[:N]


# speedup vs baseline: 8.3010x; 8.3010x over previous
import functools

import jax
import jax.numpy as jnp
from jax import lax
from jax.experimental import pallas as pl
from jax.experimental.pallas import tpu as pltpu
from jax.experimental.pallas import tpu_sc as plsc

N = 10000
E = 320000
DIN = 128
DH = 256
DOUT = 128

NP = 10240
EP = 327680
EPR = EP // 128
RPT = NP // 16
EB = 8
BR = 1024

_MESH = plsc.VectorSubcoreMesh(core_axis_name="c", subcore_axis_name="s")


def _deg_body(dst_hbm, ones_hbm, zeros_hbm, deg_out, deg_sh, dstb, onesb,
              sem):
    c = lax.axis_index("c")
    s = lax.axis_index("s")

    @pl.when(c == 0)
    def _():
        rb = s * RPT
        pltpu.sync_copy(zeros_hbm.at[pl.ds(rb, RPT)], deg_sh.at[pl.ds(rb, RPT)])
        pltpu.sync_copy(ones_hbm, onesb)
        plsc.subcore_barrier()

        ept_rows = EPR // 16
        row0 = s * ept_rows

        def blk(b, _):
            pltpu.sync_copy(dst_hbm.at[pl.ds(row0 + b * EB, EB)], dstb)

            def j_body(j, _):
                pltpu.sync_copy(onesb, deg_sh.at[dstb.at[j]], add=True)
                return 0

            return lax.fori_loop(0, EB, j_body, 0)

        lax.fori_loop(0, ept_rows // EB, blk, 0)
        plsc.subcore_barrier()
        pltpu.sync_copy(deg_sh.at[pl.ds(rb, RPT)], deg_out.at[pl.ds(rb, RPT)])


def _deg_kernel(dst2d, ones8, zeros8):
    return pl.kernel(
        _deg_body,
        out_type=jax.ShapeDtypeStruct((NP, 16), jnp.float32),
        mesh=_MESH,
        scratch_types=[
            pltpu.VMEM_SHARED((NP, 16), jnp.float32),
            pltpu.VMEM((EB, 128), jnp.int32),
            pltpu.VMEM((128, 16), jnp.float32),
            pltpu.SemaphoreType.DMA,
        ],
        compiler_params=pltpu.CompilerParams(use_tc_tiling_on_sc=False),
    )(dst2d, ones8, zeros8)


def _make_agg_body(feature_split):
    def body(table, src_hbm, dst_hbm, zeros_hbm, out, agg_sh, srcb, dstb,
             idxg, rows, gsem):
        c = lax.axis_index("c")
        s = lax.axis_index("s")
        rb = s * RPT
        pltpu.sync_copy(zeros_hbm.at[pl.ds(rb, RPT)], agg_sh.at[pl.ds(rb, RPT)])
        plsc.subcore_barrier()

        if feature_split:
            tile_rows = EPR // 16
            row0 = s * tile_rows
        else:
            tile_rows = EPR // 32
            row0 = (s * 2 + c) * tile_rows

        def blk(b, _):
            e0 = (row0 + b * EB) * 128
            pltpu.sync_copy(src_hbm.at[pl.ds(e0, EB * 128)], srcb)
            pltpu.sync_copy(dst_hbm.at[pl.ds(row0 + b * EB, EB)], dstb)

            def j_body(j, _):
                if feature_split:
                    def t_body(k, _):
                        v = srcb[pl.ds(j * 128 + k * 16, 16)]
                        idxg[pl.ds(k * 16, 16)] = v * 2 + c
                        return 0
                    lax.fori_loop(0, 8, t_body, 0)
                    gidx = idxg
                else:
                    gidx = srcb.at[pl.ds(j * 128, 128)]
                pltpu.async_copy(table.at[gidx], rows, gsem).wait()
                pltpu.sync_copy(rows, agg_sh.at[dstb.at[j]], add=True)
                return 0

            return lax.fori_loop(0, EB, j_body, 0)

        lax.fori_loop(0, tile_rows // EB, blk, 0)
        plsc.subcore_barrier()
        pltpu.sync_copy(agg_sh.at[pl.ds(rb, RPT)], out.at[c, pl.ds(rb, RPT)])

    return body


def _agg_kernel(table, src1d, dst2d, zeros128, feature_split):
    return pl.kernel(
        _make_agg_body(feature_split),
        out_type=jax.ShapeDtypeStruct((2, NP, 128), jnp.float32),
        mesh=_MESH,
        scratch_types=[
            pltpu.VMEM_SHARED((NP, 128), jnp.float32),
            pltpu.VMEM((EB * 128,), jnp.int32),
            pltpu.VMEM((EB, 128), jnp.int32),
            pltpu.VMEM((128,), jnp.int32),
            pltpu.VMEM((128, 128), jnp.float32),
            pltpu.SemaphoreType.DMA,
        ],
    )(table, src1d, dst2d, zeros128)


def _tc_matmul1(xp, W1, deg8):
    def body(x_ref, w_ref, d_ref, o_ref, dis_ref):
        dis = lax.rsqrt(d_ref[...][:, 0:1] + 1.0)
        xw = jnp.dot(x_ref[...], w_ref[...], preferred_element_type=jnp.float32)
        o_ref[...] = xw * dis
        dis_ref[...] = dis

    return pl.pallas_call(
        body,
        grid=(NP // BR,),
        in_specs=[
            pl.BlockSpec((BR, DIN), lambda i: (i, 0)),
            pl.BlockSpec((DIN, DH), lambda i: (0, 0)),
            pl.BlockSpec((BR, 16), lambda i: (i, 0)),
        ],
        out_specs=[
            pl.BlockSpec((BR, DH), lambda i: (i, 0)),
            pl.BlockSpec((BR, 1), lambda i: (i, 0)),
        ],
        out_shape=[
            jax.ShapeDtypeStruct((NP, DH), jnp.float32),
            jax.ShapeDtypeStruct((NP, 1), jnp.float32),
        ],
    )(xp, W1, deg8)


def _tc_mid(agg1, y1, dis2, b1, gam, bet, mu, var, W2):
    def body(a_ref, y_ref, d_ref, b_ref, g_ref, be_ref, m_ref, v_ref, w_ref,
             o_ref):
        a = jnp.concatenate([a_ref[0], a_ref[1]], axis=1)
        d = d_ref[...]
        g = d * (a + y_ref[...]) + b_ref[...]
        scale = g_ref[...] * lax.rsqrt(v_ref[...] + 1e-5)
        h = jnp.maximum(g * scale + (be_ref[...] - m_ref[...] * scale), 0.0)
        o_ref[...] = jnp.dot(h, w_ref[...],
                             preferred_element_type=jnp.float32) * d

    return pl.pallas_call(
        body,
        grid=(NP // BR,),
        in_specs=[
            pl.BlockSpec((2, BR, 128), lambda i: (0, i, 0)),
            pl.BlockSpec((BR, DH), lambda i: (i, 0)),
            pl.BlockSpec((BR, 1), lambda i: (i, 0)),
            pl.BlockSpec((1, DH), lambda i: (0, 0)),
            pl.BlockSpec((1, DH), lambda i: (0, 0)),
            pl.BlockSpec((1, DH), lambda i: (0, 0)),
            pl.BlockSpec((1, DH), lambda i: (0, 0)),
            pl.BlockSpec((1, DH), lambda i: (0, 0)),
            pl.BlockSpec((DH, DOUT), lambda i: (0, 0)),
        ],
        out_specs=pl.BlockSpec((BR, DOUT), lambda i: (i, 0)),
        out_shape=jax.ShapeDtypeStruct((NP, DOUT), jnp.float32),
    )(agg1, y1, dis2, b1, gam, bet, mu, var, W2)


def _tc_final(agg2, y2, dis2, b2):
    def body(a_ref, y_ref, d_ref, b_ref, o_ref):
        o_ref[...] = d_ref[...] * (a_ref[0] + a_ref[1] + y_ref[...]) + b_ref[...]

    return pl.pallas_call(
        body,
        grid=(NP // BR,),
        in_specs=[
            pl.BlockSpec((2, BR, DOUT), lambda i: (0, i, 0)),
            pl.BlockSpec((BR, DOUT), lambda i: (i, 0)),
            pl.BlockSpec((BR, 1), lambda i: (i, 0)),
            pl.BlockSpec((1, DOUT), lambda i: (0, 0)),
        ],
        out_specs=pl.BlockSpec((BR, DOUT), lambda i: (i, 0)),
        out_shape=jax.ShapeDtypeStruct((NP, DOUT), jnp.float32),
    )(agg2, y2, dis2, b2)


@jax.jit
def kernel(node_feat, edge_index, W1, b1, W2, b2, bn_gamma, bn_beta, bn_mean,
           bn_var):
    f32 = jnp.float32
    xp = jnp.zeros((NP, DIN), f32).at[:N].set(node_feat)
    pad = jnp.full((EP - E,), N, jnp.int32)
    src = jnp.concatenate([edge_index[0], pad])
    dst = jnp.concatenate([edge_index[1], pad])
    dst2d = dst.reshape(EPR, 128)

    ones8 = jnp.ones((128, 16), f32)
    zeros8 = jnp.zeros((NP, 16), f32)
    zeros128 = jnp.zeros((NP, 128), f32)

    deg8 = _deg_kernel(dst2d, ones8, zeros8)
    y1, dis2 = _tc_matmul1(xp, W1, deg8)
    agg1 = _agg_kernel(y1.reshape(2 * NP, 128), src, dst2d, zeros128,
                       feature_split=True)
    y2 = _tc_mid(agg1, y1, dis2, b1.reshape(1, DH), bn_gamma.reshape(1, DH),
                 bn_beta.reshape(1, DH), bn_mean.reshape(1, DH),
                 bn_var.reshape(1, DH), W2)
    agg2 = _agg_kernel(y2, src, dst2d, zeros128, feature_split=False)
    out = _tc_final(agg2, y2, dis2, b2.reshape(1, DOUT))
    return out[:N]


# speedup vs baseline: 9.5323x; 1.1483x over previous
import functools

import jax
import jax.numpy as jnp
from jax import lax
from jax.experimental import pallas as pl
from jax.experimental.pallas import tpu as pltpu
from jax.experimental.pallas import tpu_sc as plsc

N = 10000
E = 320000
DIN = 128
DH = 256
DOUT = 128

NP = 10240
EP = 327680
EPR = EP // 128
RPT = NP // 16
EB = 8
BR = 1024

_MESH = plsc.VectorSubcoreMesh(core_axis_name="c", subcore_axis_name="s")


def _deg_body(dst_hbm, ones_hbm, zeros_hbm, deg_out, deg_sh, dstb, onesb,
              sem):
    c = lax.axis_index("c")
    s = lax.axis_index("s")

    @pl.when(c == 0)
    def _():
        rb = s * RPT
        pltpu.sync_copy(zeros_hbm.at[pl.ds(rb, RPT)], deg_sh.at[pl.ds(rb, RPT)])
        pltpu.sync_copy(ones_hbm, onesb)
        plsc.subcore_barrier()

        ept_rows = EPR // 16
        row0 = s * ept_rows

        def blk(b, _):
            pltpu.sync_copy(dst_hbm.at[pl.ds(row0 + b * EB, EB)], dstb)

            def j_body(j, _):
                pltpu.sync_copy(onesb, deg_sh.at[dstb.at[j]], add=True)
                return 0

            return lax.fori_loop(0, EB, j_body, 0)

        lax.fori_loop(0, ept_rows // EB, blk, 0)
        plsc.subcore_barrier()
        pltpu.sync_copy(deg_sh.at[pl.ds(rb, RPT)], deg_out.at[pl.ds(rb, RPT)])


def _deg_kernel(dst2d, ones8, zeros8):
    return pl.kernel(
        _deg_body,
        out_type=jax.ShapeDtypeStruct((NP, 16), jnp.float32),
        mesh=_MESH,
        scratch_types=[
            pltpu.VMEM_SHARED((NP, 16), jnp.float32),
            pltpu.VMEM((EB, 128), jnp.int32),
            pltpu.VMEM((128, 16), jnp.float32),
            pltpu.SemaphoreType.DMA,
        ],
        compiler_params=pltpu.CompilerParams(use_tc_tiling_on_sc=False),
    )(dst2d, ones8, zeros8)


def _make_agg_body(feature_split):
    def body(table, src_hbm, dst_hbm, zeros_hbm, out, agg_sh, srcA, srcB,
             dstA, dstB, r0, r1, gsem, ssem, isem):
        c = lax.axis_index("c")
        s = lax.axis_index("s")
        rows = [r0, r1]
        rb = s * RPT
        pltpu.sync_copy(zeros_hbm.at[pl.ds(rb, RPT)], agg_sh.at[pl.ds(rb, RPT)])
        plsc.subcore_barrier()

        if feature_split:
            nchunks = EPR // 16
            row0 = s * nchunks
            tbl = table.at[c]
        else:
            nchunks = EPR // 32
            row0 = (s * 2 + c) * nchunks
            tbl = table
        nbody = nchunks // 16

        def drain_one(sem, k):
            pltpu.make_async_copy(zeros_hbm.at[pl.ds(0, 128)], rows[k],
                                  sem).wait()

        def issue_gather(m, srcbuf, r_local, k):
            @pl.when(m < nchunks)
            def _():
                @pl.when(m >= 2)
                def _():
                    drain_one(ssem, k)
                pltpu.async_copy(tbl.at[srcbuf.at[r_local]], rows[k], gsem)

        def process(dstbuf, r_local, k):
            drain_one(gsem, k)
            pltpu.async_copy(rows[k], agg_sh.at[dstbuf.at[r_local]], ssem,
                             add=True)

        pltpu.sync_copy(src_hbm.at[pl.ds(row0, EB)], srcA)
        pltpu.sync_copy(dst_hbm.at[pl.ds(row0, EB)], dstA)
        pltpu.async_copy(tbl.at[srcA.at[0]], rows[0], gsem)

        def fbody(h, _):
            base = row0 + h * 16
            m0 = h * 16
            desc_b = desc_a = None
            for ml in range(16):
                if ml == 7:
                    desc_b[0].wait()
                    desc_b[1].wait()
                if ml == 15:
                    desc_a[0].wait()
                    desc_a[1].wait()
                dstbuf = dstA if ml < 8 else dstB
                process(dstbuf, ml % 8, ml % 2)
                kk = ml + 1
                srcbuf = srcA if (kk < 8 or kk >= 16) else srcB
                issue_gather(m0 + kk, srcbuf, kk % 8, kk % 2)
                if ml == 1:
                    desc_b = (
                        pltpu.async_copy(src_hbm.at[pl.ds(base + 8, EB)],
                                         srcB, isem),
                        pltpu.async_copy(dst_hbm.at[pl.ds(base + 8, EB)],
                                         dstB, isem),
                    )
                if ml == 8:
                    nb = jnp.minimum(base + 16, EPR - EB)
                    desc_a = (
                        pltpu.async_copy(src_hbm.at[pl.ds(nb, EB)], srcA,
                                         isem),
                        pltpu.async_copy(dst_hbm.at[pl.ds(nb, EB)], dstA,
                                         isem),
                    )
            return 0

        lax.fori_loop(0, nbody, fbody, 0)
        for k in range(2):
            drain_one(ssem, k)
        plsc.subcore_barrier()
        pltpu.sync_copy(agg_sh.at[pl.ds(rb, RPT)], out.at[c, pl.ds(rb, RPT)])

    return body


def _agg_kernel(table, src2d, dst2d, zeros128, feature_split):
    return pl.kernel(
        _make_agg_body(feature_split),
        out_type=jax.ShapeDtypeStruct((2, NP, 128), jnp.float32),
        mesh=_MESH,
        scratch_types=[
            pltpu.VMEM_SHARED((NP, 128), jnp.float32),
            pltpu.VMEM((EB, 128), jnp.int32),
            pltpu.VMEM((EB, 128), jnp.int32),
            pltpu.VMEM((EB, 128), jnp.int32),
            pltpu.VMEM((EB, 128), jnp.int32),
            pltpu.VMEM((128, 128), jnp.float32),
            pltpu.VMEM((128, 128), jnp.float32),
            pltpu.SemaphoreType.DMA,
            pltpu.SemaphoreType.DMA,
            pltpu.SemaphoreType.DMA,
        ],
    )(table, src2d, dst2d, zeros128)


def _tc_matmul1(xp, W1, deg8):
    def body(x_ref, w_ref, d_ref, o_ref, dis_ref):
        dis = lax.rsqrt(d_ref[...][:, 0:1] + 1.0)
        xw = jnp.dot(x_ref[...], w_ref[...], preferred_element_type=jnp.float32)
        y = xw * dis
        o_ref[0] = y[:, :128]
        o_ref[1] = y[:, 128:]
        dis_ref[...] = dis

    return pl.pallas_call(
        body,
        grid=(NP // BR,),
        in_specs=[
            pl.BlockSpec((BR, DIN), lambda i: (i, 0)),
            pl.BlockSpec((DIN, DH), lambda i: (0, 0)),
            pl.BlockSpec((BR, 16), lambda i: (i, 0)),
        ],
        out_specs=[
            pl.BlockSpec((2, BR, 128), lambda i: (0, i, 0)),
            pl.BlockSpec((BR, 1), lambda i: (i, 0)),
        ],
        out_shape=[
            jax.ShapeDtypeStruct((2, NP, 128), jnp.float32),
            jax.ShapeDtypeStruct((NP, 1), jnp.float32),
        ],
    )(xp, W1, deg8)


def _tc_mid(agg1, y1, dis2, b1, gam, bet, mu, var, W2):
    def body(a_ref, y_ref, d_ref, b_ref, g_ref, be_ref, m_ref, v_ref, w_ref,
             o_ref):
        a = jnp.concatenate([a_ref[0], a_ref[1]], axis=1)
        y = jnp.concatenate([y_ref[0], y_ref[1]], axis=1)
        d = d_ref[...]
        g = d * (a + y) + b_ref[...]
        scale = g_ref[...] * lax.rsqrt(v_ref[...] + 1e-5)
        h = jnp.maximum(g * scale + (be_ref[...] - m_ref[...] * scale), 0.0)
        o_ref[...] = jnp.dot(h, w_ref[...],
                             preferred_element_type=jnp.float32) * d

    return pl.pallas_call(
        body,
        grid=(NP // BR,),
        in_specs=[
            pl.BlockSpec((2, BR, 128), lambda i: (0, i, 0)),
            pl.BlockSpec((2, BR, 128), lambda i: (0, i, 0)),
            pl.BlockSpec((BR, 1), lambda i: (i, 0)),
            pl.BlockSpec((1, DH), lambda i: (0, 0)),
            pl.BlockSpec((1, DH), lambda i: (0, 0)),
            pl.BlockSpec((1, DH), lambda i: (0, 0)),
            pl.BlockSpec((1, DH), lambda i: (0, 0)),
            pl.BlockSpec((1, DH), lambda i: (0, 0)),
            pl.BlockSpec((DH, DOUT), lambda i: (0, 0)),
        ],
        out_specs=pl.BlockSpec((BR, DOUT), lambda i: (i, 0)),
        out_shape=jax.ShapeDtypeStruct((NP, DOUT), jnp.float32),
    )(agg1, y1, dis2, b1, gam, bet, mu, var, W2)


def _tc_final(agg2, y2, dis2, b2):
    def body(a_ref, y_ref, d_ref, b_ref, o_ref):
        o_ref[...] = d_ref[...] * (a_ref[0] + a_ref[1] + y_ref[...]) + b_ref[...]

    return pl.pallas_call(
        body,
        grid=(NP // BR,),
        in_specs=[
            pl.BlockSpec((2, BR, DOUT), lambda i: (0, i, 0)),
            pl.BlockSpec((BR, DOUT), lambda i: (i, 0)),
            pl.BlockSpec((BR, 1), lambda i: (i, 0)),
            pl.BlockSpec((1, DOUT), lambda i: (0, 0)),
        ],
        out_specs=pl.BlockSpec((BR, DOUT), lambda i: (i, 0)),
        out_shape=jax.ShapeDtypeStruct((NP, DOUT), jnp.float32),
    )(agg2, y2, dis2, b2)


@jax.jit
def kernel(node_feat, edge_index, W1, b1, W2, b2, bn_gamma, bn_beta, bn_mean,
           bn_var):
    f32 = jnp.float32
    xp = jnp.zeros((NP, DIN), f32).at[:N].set(node_feat)
    pad = jnp.full((EP - E,), N, jnp.int32)
    src = jnp.concatenate([edge_index[0], pad])
    dst = jnp.concatenate([edge_index[1], pad])
    src2d = src.reshape(EPR, 128)
    dst2d = dst.reshape(EPR, 128)

    ones8 = jnp.ones((128, 16), f32)
    zeros8 = jnp.zeros((NP, 16), f32)
    zeros128 = jnp.zeros((NP, 128), f32)

    deg8 = _deg_kernel(dst2d, ones8, zeros8)
    y1p, dis2 = _tc_matmul1(xp, W1, deg8)
    agg1 = _agg_kernel(y1p, src2d, dst2d, zeros128, feature_split=True)
    y2 = _tc_mid(agg1, y1p, dis2, b1.reshape(1, DH), bn_gamma.reshape(1, DH),
                 bn_beta.reshape(1, DH), bn_mean.reshape(1, DH),
                 bn_var.reshape(1, DH), W2)
    agg2 = _agg_kernel(y2, src2d, dst2d, zeros128, feature_split=False)
    out = _tc_final(agg2, y2, dis2, b2.reshape(1, DOUT))
    return out[:N]


# speedup vs baseline: 10.1780x; 1.0677x over previous
import functools

import jax
import jax.numpy as jnp
from jax import lax
from jax.experimental import pallas as pl
from jax.experimental.pallas import tpu as pltpu
from jax.experimental.pallas import tpu_sc as plsc

N = 10000
E = 320000
DIN = 128
DH = 256
DOUT = 128

NP = 10240
EP = 327680
EPR = EP // 128
RPT = NP // 16
EB = 8
BR = 1024

_MESH = plsc.VectorSubcoreMesh(core_axis_name="c", subcore_axis_name="s")


def _deg_body(dst_hbm, ones_hbm, zeros_hbm, deg_out, deg_sh, dstb, onesb,
              sem):
    c = lax.axis_index("c")
    s = lax.axis_index("s")

    @pl.when(c == 0)
    def _():
        rb = s * RPT
        pltpu.sync_copy(zeros_hbm.at[pl.ds(rb, RPT)], deg_sh.at[pl.ds(rb, RPT)])
        pltpu.sync_copy(ones_hbm, onesb)
        plsc.subcore_barrier()

        ept_rows = EPR // 16
        row0 = s * ept_rows

        def blk(b, _):
            pltpu.sync_copy(dst_hbm.at[pl.ds(row0 + b * EB, EB)], dstb)

            def j_body(j, _):
                pltpu.sync_copy(onesb, deg_sh.at[dstb.at[j]], add=True)
                return 0

            return lax.fori_loop(0, EB, j_body, 0)

        lax.fori_loop(0, ept_rows // EB, blk, 0)
        plsc.subcore_barrier()
        pltpu.sync_copy(deg_sh.at[pl.ds(rb, RPT)], deg_out.at[pl.ds(rb, RPT)])


def _deg_kernel(dst2d, ones8, zeros8):
    return pl.kernel(
        _deg_body,
        out_type=jax.ShapeDtypeStruct((NP, 16), jnp.float32),
        mesh=_MESH,
        scratch_types=[
            pltpu.VMEM_SHARED((NP, 16), jnp.float32),
            pltpu.VMEM((EB, 128), jnp.int32),
            pltpu.VMEM((128, 16), jnp.float32),
            pltpu.SemaphoreType.DMA,
        ],
        compiler_params=pltpu.CompilerParams(use_tc_tiling_on_sc=False),
    )(dst2d, ones8, zeros8)


def _agg_body(table, src_hbm, dst_hbm, zeros_hbm, out, agg_sh, srcA, srcB,
              dstA, dstB, r0, r1, gsem, ssem, isem):
    c = lax.axis_index("c")
    s = lax.axis_index("s")
    rows = [r0, r1]
    rb = s * RPT
    pltpu.sync_copy(zeros_hbm.at[pl.ds(rb, RPT)], agg_sh.at[pl.ds(rb, RPT)])
    plsc.subcore_barrier()

    nchunks = EPR // 32
    row0 = (s * 2 + c) * nchunks
    nbody = nchunks // 16

    def drain_one(sem, k):
        pltpu.make_async_copy(zeros_hbm.at[pl.ds(0, 128)], rows[k],
                              sem).wait()

    def issue_gather(m, srcbuf, r_local, k):
        @pl.when(m < nchunks)
        def _():
            @pl.when(m >= 2)
            def _():
                drain_one(ssem, k)
            pltpu.async_copy(table.at[srcbuf.at[r_local]], rows[k], gsem)

    def process(dstbuf, r_local, k):
        drain_one(gsem, k)
        pltpu.async_copy(rows[k], agg_sh.at[dstbuf.at[r_local]], ssem,
                         add=True)

    pltpu.sync_copy(src_hbm.at[pl.ds(row0, EB)], srcA)
    pltpu.sync_copy(dst_hbm.at[pl.ds(row0, EB)], dstA)
    pltpu.async_copy(table.at[srcA.at[0]], rows[0], gsem)

    def fbody(h, _):
        base = row0 + h * 16
        m0 = h * 16
        desc_b = desc_a = None
        for ml in range(16):
            if ml == 7:
                desc_b[0].wait()
                desc_b[1].wait()
            if ml == 15:
                desc_a[0].wait()
                desc_a[1].wait()
            dstbuf = dstA if ml < 8 else dstB
            process(dstbuf, ml % 8, ml % 2)
            kk = ml + 1
            srcbuf = srcA if (kk < 8 or kk >= 16) else srcB
            issue_gather(m0 + kk, srcbuf, kk % 8, kk % 2)
            if ml == 1:
                desc_b = (
                    pltpu.async_copy(src_hbm.at[pl.ds(base + 8, EB)],
                                     srcB, isem),
                    pltpu.async_copy(dst_hbm.at[pl.ds(base + 8, EB)],
                                     dstB, isem),
                )
            if ml == 8:
                nb = jnp.minimum(base + 16, EPR - EB)
                desc_a = (
                    pltpu.async_copy(src_hbm.at[pl.ds(nb, EB)], srcA,
                                     isem),
                    pltpu.async_copy(dst_hbm.at[pl.ds(nb, EB)], dstA,
                                     isem),
                )
        return 0

    lax.fori_loop(0, nbody, fbody, 0)
    for k in range(2):
        drain_one(ssem, k)
    plsc.subcore_barrier()
    pltpu.sync_copy(agg_sh.at[pl.ds(rb, RPT)], out.at[c, pl.ds(rb, RPT)])


def _agg_kernel(table, src2d, dst2d, zeros128):
    return pl.kernel(
        _agg_body,
        out_type=jax.ShapeDtypeStruct((2, NP, 128), jnp.float32),
        mesh=_MESH,
        scratch_types=[
            pltpu.VMEM_SHARED((NP, 128), jnp.float32),
            pltpu.VMEM((EB, 128), jnp.int32),
            pltpu.VMEM((EB, 128), jnp.int32),
            pltpu.VMEM((EB, 128), jnp.int32),
            pltpu.VMEM((EB, 128), jnp.int32),
            pltpu.VMEM((128, 128), jnp.float32),
            pltpu.VMEM((128, 128), jnp.float32),
            pltpu.SemaphoreType.DMA,
            pltpu.SemaphoreType.DMA,
            pltpu.SemaphoreType.DMA,
        ],
    )(table, src2d, dst2d, zeros128)


def _tc_scale(xp, deg8):
    def body(x_ref, d_ref, t_ref, dis_ref):
        dis = lax.rsqrt(d_ref[...][:, 0:1] + 1.0)
        t_ref[...] = x_ref[...] * dis
        dis_ref[...] = dis

    return pl.pallas_call(
        body,
        grid=(NP // BR,),
        in_specs=[
            pl.BlockSpec((BR, DIN), lambda i: (i, 0)),
            pl.BlockSpec((BR, 16), lambda i: (i, 0)),
        ],
        out_specs=[
            pl.BlockSpec((BR, DIN), lambda i: (i, 0)),
            pl.BlockSpec((BR, 1), lambda i: (i, 0)),
        ],
        out_shape=[
            jax.ShapeDtypeStruct((NP, DIN), jnp.float32),
            jax.ShapeDtypeStruct((NP, 1), jnp.float32),
        ],
    )(xp, deg8)


def _tc_mid(aggx, t, dis2, W1, b1, gam, bet, mu, var, W2):
    def body(a_ref, t_ref, d_ref, w1_ref, b_ref, g_ref, be_ref, m_ref, v_ref,
             w2_ref, o_ref):
        sm = a_ref[0] + a_ref[1] + t_ref[...]
        u = jnp.dot(sm, w1_ref[...], preferred_element_type=jnp.float32)
        d = d_ref[...]
        g = d * u + b_ref[...]
        scale = g_ref[...] * lax.rsqrt(v_ref[...] + 1e-5)
        h = jnp.maximum(g * scale + (be_ref[...] - m_ref[...] * scale), 0.0)
        o_ref[...] = jnp.dot(h, w2_ref[...],
                             preferred_element_type=jnp.float32) * d

    return pl.pallas_call(
        body,
        grid=(NP // BR,),
        in_specs=[
            pl.BlockSpec((2, BR, 128), lambda i: (0, i, 0)),
            pl.BlockSpec((BR, DIN), lambda i: (i, 0)),
            pl.BlockSpec((BR, 1), lambda i: (i, 0)),
            pl.BlockSpec((DIN, DH), lambda i: (0, 0)),
            pl.BlockSpec((1, DH), lambda i: (0, 0)),
            pl.BlockSpec((1, DH), lambda i: (0, 0)),
            pl.BlockSpec((1, DH), lambda i: (0, 0)),
            pl.BlockSpec((1, DH), lambda i: (0, 0)),
            pl.BlockSpec((1, DH), lambda i: (0, 0)),
            pl.BlockSpec((DH, DOUT), lambda i: (0, 0)),
        ],
        out_specs=pl.BlockSpec((BR, DOUT), lambda i: (i, 0)),
        out_shape=jax.ShapeDtypeStruct((NP, DOUT), jnp.float32),
    )(aggx, t, dis2, W1, b1, gam, bet, mu, var, W2)


def _tc_final(agg2, y2, dis2, b2):
    def body(a_ref, y_ref, d_ref, b_ref, o_ref):
        o_ref[...] = d_ref[...] * (a_ref[0] + a_ref[1] + y_ref[...]) + b_ref[...]

    return pl.pallas_call(
        body,
        grid=(NP // BR,),
        in_specs=[
            pl.BlockSpec((2, BR, DOUT), lambda i: (0, i, 0)),
            pl.BlockSpec((BR, DOUT), lambda i: (i, 0)),
            pl.BlockSpec((BR, 1), lambda i: (i, 0)),
            pl.BlockSpec((1, DOUT), lambda i: (0, 0)),
        ],
        out_specs=pl.BlockSpec((BR, DOUT), lambda i: (i, 0)),
        out_shape=jax.ShapeDtypeStruct((NP, DOUT), jnp.float32),
    )(agg2, y2, dis2, b2)


@jax.jit
def kernel(node_feat, edge_index, W1, b1, W2, b2, bn_gamma, bn_beta, bn_mean,
           bn_var):
    f32 = jnp.float32
    xp = jnp.zeros((NP, DIN), f32).at[:N].set(node_feat)
    pad = jnp.full((EP - E,), N, jnp.int32)
    src = jnp.concatenate([edge_index[0], pad])
    dst = jnp.concatenate([edge_index[1], pad])
    src2d = src.reshape(EPR, 128)
    dst2d = dst.reshape(EPR, 128)

    ones8 = jnp.ones((128, 16), f32)
    zeros8 = jnp.zeros((NP, 16), f32)
    zeros128 = jnp.zeros((NP, 128), f32)

    deg8 = _deg_kernel(dst2d, ones8, zeros8)
    t, dis2 = _tc_scale(xp, deg8)
    aggx = _agg_kernel(t, src2d, dst2d, zeros128)
    y2 = _tc_mid(aggx, t, dis2, W1, b1.reshape(1, DH), bn_gamma.reshape(1, DH),
                 bn_beta.reshape(1, DH), bn_mean.reshape(1, DH),
                 bn_var.reshape(1, DH), W2)
    agg2 = _agg_kernel(y2, src2d, dst2d, zeros128)
    out = _tc_final(agg2, y2, dis2, b2.reshape(1, DOUT))
    return out[:N]


# speedup vs baseline: 10.6929x; 1.0506x over previous
import functools

import jax
import jax.numpy as jnp
from jax import lax
from jax.experimental import pallas as pl
from jax.experimental.pallas import tpu as pltpu
from jax.experimental.pallas import tpu_sc as plsc

N = 10000
E = 320000
DIN = 128
DH = 256
DOUT = 128

NP = 10240
EP = 327680
EPR = EP // 128
RPT = NP // 16
EB = 8
BR = 1024

_MESH = plsc.VectorSubcoreMesh(core_axis_name="c", subcore_axis_name="s")


def _deg_body(dst_hbm, ones_hbm, zeros_hbm, deg_out, deg_sh, dstb, onesb,
              sem):
    c = lax.axis_index("c")
    s = lax.axis_index("s")

    @pl.when(c == 0)
    def _():
        rb = s * RPT
        pltpu.sync_copy(zeros_hbm.at[pl.ds(rb, RPT)], deg_sh.at[pl.ds(rb, RPT)])
        pltpu.sync_copy(ones_hbm, onesb)
        plsc.subcore_barrier()

        ept_rows = EPR // 16
        row0 = s * ept_rows

        def blk(b, _):
            pltpu.sync_copy(dst_hbm.at[pl.ds(row0 + b * EB, EB)], dstb)

            def j_body(j, _):
                pltpu.sync_copy(onesb, deg_sh.at[dstb.at[j]], add=True)
                return 0

            return lax.fori_loop(0, EB, j_body, 0)

        lax.fori_loop(0, ept_rows // EB, blk, 0)
        plsc.subcore_barrier()
        pltpu.sync_copy(deg_sh.at[pl.ds(rb, RPT)], deg_out.at[pl.ds(rb, RPT)])


def _deg_kernel(dst2d, ones8, zeros8):
    return pl.kernel(
        _deg_body,
        out_type=jax.ShapeDtypeStruct((NP, 16), jnp.float32),
        mesh=_MESH,
        scratch_types=[
            pltpu.VMEM_SHARED((NP, 16), jnp.float32),
            pltpu.VMEM((EB, 128), jnp.int32),
            pltpu.VMEM((128, 16), jnp.float32),
            pltpu.SemaphoreType.DMA,
        ],
        compiler_params=pltpu.CompilerParams(use_tc_tiling_on_sc=False),
    )(dst2d, ones8, zeros8)


def _agg_body(table, src_hbm, dst_hbm, zeros_hbm, out, agg_sh, srcA, srcB,
              dstA, dstB, r0, r1, gsem, ssem, isem):
    c = lax.axis_index("c")
    s = lax.axis_index("s")
    rows = [r0, r1]
    rb = s * RPT
    pltpu.sync_copy(zeros_hbm.at[pl.ds(rb, RPT)], agg_sh.at[pl.ds(rb, RPT)])
    plsc.subcore_barrier()

    nchunks = EPR // 32
    row0 = (s * 2 + c) * nchunks
    nbody = nchunks // 16
    tbl = table.at[c]

    def drain_one(sem, k):
        pltpu.make_async_copy(zeros_hbm.at[pl.ds(0, 128)], rows[k],
                              sem).wait()

    def issue_gather(m, srcbuf, r_local, k):
        @pl.when(m < nchunks)
        def _():
            @pl.when(m >= 2)
            def _():
                drain_one(ssem, k)
            pltpu.async_copy(tbl.at[srcbuf.at[r_local]], rows[k], gsem)

    def process(dstbuf, r_local, k):
        drain_one(gsem, k)
        pltpu.async_copy(rows[k], agg_sh.at[dstbuf.at[r_local]], ssem,
                         add=True)

    pltpu.sync_copy(src_hbm.at[pl.ds(row0, EB)], srcA)
    pltpu.sync_copy(dst_hbm.at[pl.ds(row0, EB)], dstA)
    pltpu.async_copy(tbl.at[srcA.at[0]], rows[0], gsem)

    def fbody(h, _):
        base = row0 + h * 16
        m0 = h * 16
        desc_b = desc_a = None
        for ml in range(16):
            if ml == 7:
                desc_b[0].wait()
                desc_b[1].wait()
            if ml == 15:
                desc_a[0].wait()
                desc_a[1].wait()
            dstbuf = dstA if ml < 8 else dstB
            process(dstbuf, ml % 8, ml % 2)
            kk = ml + 1
            srcbuf = srcA if (kk < 8 or kk >= 16) else srcB
            issue_gather(m0 + kk, srcbuf, kk % 8, kk % 2)
            if ml == 1:
                desc_b = (
                    pltpu.async_copy(src_hbm.at[pl.ds(base + 8, EB)],
                                     srcB, isem),
                    pltpu.async_copy(dst_hbm.at[pl.ds(base + 8, EB)],
                                     dstB, isem),
                )
            if ml == 8:
                nb = jnp.minimum(base + 16, EPR - EB)
                desc_a = (
                    pltpu.async_copy(src_hbm.at[pl.ds(nb, EB)], srcA,
                                     isem),
                    pltpu.async_copy(dst_hbm.at[pl.ds(nb, EB)], dstA,
                                     isem),
                )
        return 0

    lax.fori_loop(0, nbody, fbody, 0)
    for k in range(2):
        drain_one(ssem, k)
    plsc.subcore_barrier()
    pltpu.sync_copy(agg_sh.at[pl.ds(rb, RPT)], out.at[c, pl.ds(rb, RPT)])


def _agg_kernel(table, src2d, dst2d, zeros128):
    return pl.kernel(
        _agg_body,
        out_type=jax.ShapeDtypeStruct((2, NP, 128), jnp.float32),
        mesh=_MESH,
        scratch_types=[
            pltpu.VMEM_SHARED((NP, 128), jnp.float32),
            pltpu.VMEM((EB, 128), jnp.int32),
            pltpu.VMEM((EB, 128), jnp.int32),
            pltpu.VMEM((EB, 128), jnp.int32),
            pltpu.VMEM((EB, 128), jnp.int32),
            pltpu.VMEM((128, 128), jnp.float32),
            pltpu.VMEM((128, 128), jnp.float32),
            pltpu.SemaphoreType.DMA,
            pltpu.SemaphoreType.DMA,
            pltpu.SemaphoreType.DMA,
        ],
    )(table, src2d, dst2d, zeros128)


def _tc_scale(xp, deg8):
    def body(x_ref, d_ref, t_ref, dis_ref):
        dis = lax.rsqrt(d_ref[...][:, 0:1] + 1.0)
        t = x_ref[...] * dis
        t_ref[0] = t
        t_ref[1] = t
        dis_ref[...] = dis

    return pl.pallas_call(
        body,
        grid=(NP // BR,),
        in_specs=[
            pl.BlockSpec((BR, DIN), lambda i: (i, 0)),
            pl.BlockSpec((BR, 16), lambda i: (i, 0)),
        ],
        out_specs=[
            pl.BlockSpec((2, BR, DIN), lambda i: (0, i, 0)),
            pl.BlockSpec((BR, 1), lambda i: (i, 0)),
        ],
        out_shape=[
            jax.ShapeDtypeStruct((2, NP, DIN), jnp.float32),
            jax.ShapeDtypeStruct((NP, 1), jnp.float32),
        ],
    )(xp, deg8)


def _tc_mid(aggx, t, dis2, W1, b1, gam, bet, mu, var, W2):
    def body(a_ref, t_ref, d_ref, w1_ref, b_ref, g_ref, be_ref, m_ref, v_ref,
             w2_ref, o_ref):
        sm = a_ref[0] + a_ref[1] + t_ref[0]
        u = jnp.dot(sm, w1_ref[...], preferred_element_type=jnp.float32)
        d = d_ref[...]
        g = d * u + b_ref[...]
        scale = g_ref[...] * lax.rsqrt(v_ref[...] + 1e-5)
        h = jnp.maximum(g * scale + (be_ref[...] - m_ref[...] * scale), 0.0)
        y2 = jnp.dot(h, w2_ref[...], preferred_element_type=jnp.float32) * d
        o_ref[0] = y2
        o_ref[1] = y2

    return pl.pallas_call(
        body,
        grid=(NP // BR,),
        in_specs=[
            pl.BlockSpec((2, BR, 128), lambda i: (0, i, 0)),
            pl.BlockSpec((2, BR, DIN), lambda i: (0, i, 0)),
            pl.BlockSpec((BR, 1), lambda i: (i, 0)),
            pl.BlockSpec((DIN, DH), lambda i: (0, 0)),
            pl.BlockSpec((1, DH), lambda i: (0, 0)),
            pl.BlockSpec((1, DH), lambda i: (0, 0)),
            pl.BlockSpec((1, DH), lambda i: (0, 0)),
            pl.BlockSpec((1, DH), lambda i: (0, 0)),
            pl.BlockSpec((1, DH), lambda i: (0, 0)),
            pl.BlockSpec((DH, DOUT), lambda i: (0, 0)),
        ],
        out_specs=pl.BlockSpec((2, BR, DOUT), lambda i: (0, i, 0)),
        out_shape=jax.ShapeDtypeStruct((2, NP, DOUT), jnp.float32),
    )(aggx, t, dis2, W1, b1, gam, bet, mu, var, W2)


def _tc_final(agg2, y2, dis2, b2):
    def body(a_ref, y_ref, d_ref, b_ref, o_ref):
        o_ref[...] = d_ref[...] * (a_ref[0] + a_ref[1] + y_ref[0]) + b_ref[...]

    return pl.pallas_call(
        body,
        grid=(NP // BR,),
        in_specs=[
            pl.BlockSpec((2, BR, DOUT), lambda i: (0, i, 0)),
            pl.BlockSpec((2, BR, DOUT), lambda i: (0, i, 0)),
            pl.BlockSpec((BR, 1), lambda i: (i, 0)),
            pl.BlockSpec((1, DOUT), lambda i: (0, 0)),
        ],
        out_specs=pl.BlockSpec((BR, DOUT), lambda i: (i, 0)),
        out_shape=jax.ShapeDtypeStruct((NP, DOUT), jnp.float32),
    )(agg2, y2, dis2, b2)


@jax.jit
def kernel(node_feat, edge_index, W1, b1, W2, b2, bn_gamma, bn_beta, bn_mean,
           bn_var):
    f32 = jnp.float32
    xp = jnp.zeros((NP, DIN), f32).at[:N].set(node_feat)
    pad = jnp.full((EP - E,), N, jnp.int32)
    src = jnp.concatenate([edge_index[0], pad])
    dst = jnp.concatenate([edge_index[1], pad])
    src2d = src.reshape(EPR, 128)
    dst2d = dst.reshape(EPR, 128)

    ones8 = jnp.ones((128, 16), f32)
    zeros8 = jnp.zeros((NP, 16), f32)
    zeros128 = jnp.zeros((NP, 128), f32)

    deg8 = _deg_kernel(dst2d, ones8, zeros8)
    t, dis2 = _tc_scale(xp, deg8)
    aggx = _agg_kernel(t, src2d, dst2d, zeros128)
    y2 = _tc_mid(aggx, t, dis2, W1, b1.reshape(1, DH), bn_gamma.reshape(1, DH),
                 bn_beta.reshape(1, DH), bn_mean.reshape(1, DH),
                 bn_var.reshape(1, DH), W2)
    agg2 = _agg_kernel(y2, src2d, dst2d, zeros128)
    out = _tc_final(agg2, y2, dis2, b2.reshape(1, DOUT))
    return out[:N]


# speedup vs baseline: 11.9738x; 1.1198x over previous
import functools

import jax
import jax.numpy as jnp
from jax import lax
from jax.experimental import pallas as pl
from jax.experimental.pallas import tpu as pltpu
from jax.experimental.pallas import tpu_sc as plsc

N = 10000
E = 320000
DIN = 128
DH = 256
DOUT = 128

NP = 10240
EP = 327680
EPR = EP // 128
RPT = NP // 16
EB = 8
BR = 1024
NCH0 = 128
NCH1 = 32

_MESH = plsc.VectorSubcoreMesh(core_axis_name="c", subcore_axis_name="s")


def _deg_body(dst_hbm, ones_hbm, zeros_hbm, deg_out, deg_sh, dstb, onesb,
              sem):
    c = lax.axis_index("c")
    s = lax.axis_index("s")

    @pl.when(c == 0)
    def _():
        rb = s * RPT
        pltpu.sync_copy(zeros_hbm.at[pl.ds(rb, RPT)], deg_sh.at[pl.ds(rb, RPT)])
        pltpu.sync_copy(ones_hbm, onesb)
        plsc.subcore_barrier()

        ept_rows = EPR // 16
        row0 = s * ept_rows

        def blk(b, _):
            pltpu.sync_copy(dst_hbm.at[pl.ds(row0 + b * EB, EB)], dstb)

            def j_body(j, _):
                pltpu.sync_copy(onesb, deg_sh.at[dstb.at[j]], add=True)
                return 0

            return lax.fori_loop(0, EB, j_body, 0)

        lax.fori_loop(0, ept_rows // EB, blk, 0)
        plsc.subcore_barrier()
        pltpu.sync_copy(deg_sh.at[pl.ds(rb, RPT)], deg_out.at[pl.ds(rb, RPT)])


def _deg_kernel(dst2d, ones8, zeros8):
    return pl.kernel(
        _deg_body,
        out_type=jax.ShapeDtypeStruct((NP, 16), jnp.float32),
        mesh=_MESH,
        scratch_types=[
            pltpu.VMEM_SHARED((NP, 16), jnp.float32),
            pltpu.VMEM((EB, 128), jnp.int32),
            pltpu.VMEM((128, 16), jnp.float32),
            pltpu.SemaphoreType.DMA,
        ],
        compiler_params=pltpu.CompilerParams(use_tc_tiling_on_sc=False),
    )(dst2d, ones8, zeros8)


def _agg_body(table, src_hbm, dst_hbm, zeros_hbm, out, agg_sh, srcA, srcB,
              dstA, dstB, r0, r1, gsem, ssem, isem):
    c = lax.axis_index("c")
    s = lax.axis_index("s")
    rows = [r0, r1]
    rb = s * RPT
    pltpu.sync_copy(zeros_hbm.at[pl.ds(rb, RPT)], agg_sh.at[pl.ds(rb, RPT)])
    plsc.subcore_barrier()

    is0 = c == 0
    nchunks = jnp.where(is0, NCH0, NCH1)
    row0 = jnp.where(is0, s * NCH0, 16 * NCH0 + s * NCH1)
    nbody = nchunks // 16
    tbl = table.at[c]

    def drain_one(sem, k):
        pltpu.make_async_copy(zeros_hbm.at[pl.ds(0, 128)], rows[k],
                              sem).wait()

    def issue_gather(m, srcbuf, r_local, k):
        @pl.when(m < nchunks)
        def _():
            @pl.when(m >= 2)
            def _():
                drain_one(ssem, k)
            pltpu.async_copy(tbl.at[srcbuf.at[r_local]], rows[k], gsem)

    def process(dstbuf, r_local, k):
        drain_one(gsem, k)
        pltpu.async_copy(rows[k], agg_sh.at[dstbuf.at[r_local]], ssem,
                         add=True)

    pltpu.sync_copy(src_hbm.at[pl.ds(row0, EB)], srcA)
    pltpu.sync_copy(dst_hbm.at[pl.ds(row0, EB)], dstA)
    pltpu.async_copy(tbl.at[srcA.at[0]], rows[0], gsem)

    def fbody(h, _):
        base = row0 + h * 16
        m0 = h * 16
        desc_b = desc_a = None
        for ml in range(16):
            if ml == 7:
                desc_b[0].wait()
                desc_b[1].wait()
            if ml == 15:
                desc_a[0].wait()
                desc_a[1].wait()
            dstbuf = dstA if ml < 8 else dstB
            process(dstbuf, ml % 8, ml % 2)
            kk = ml + 1
            srcbuf = srcA if (kk < 8 or kk >= 16) else srcB
            issue_gather(m0 + kk, srcbuf, kk % 8, kk % 2)
            if ml == 1:
                desc_b = (
                    pltpu.async_copy(src_hbm.at[pl.ds(base + 8, EB)],
                                     srcB, isem),
                    pltpu.async_copy(dst_hbm.at[pl.ds(base + 8, EB)],
                                     dstB, isem),
                )
            if ml == 8:
                nb = jnp.minimum(base + 16, EPR - EB)
                desc_a = (
                    pltpu.async_copy(src_hbm.at[pl.ds(nb, EB)], srcA,
                                     isem),
                    pltpu.async_copy(dst_hbm.at[pl.ds(nb, EB)], dstA,
                                     isem),
                )
        return 0

    lax.fori_loop(0, nbody, fbody, 0)
    for k in range(2):
        drain_one(ssem, k)
    plsc.subcore_barrier()
    pltpu.sync_copy(agg_sh.at[pl.ds(rb, RPT)], out.at[c, pl.ds(rb, RPT)])


def _agg_kernel(table, src2d, dst2d, zeros128):
    return pl.kernel(
        _agg_body,
        out_type=jax.ShapeDtypeStruct((2, NP, 128), jnp.float32),
        mesh=_MESH,
        scratch_types=[
            pltpu.VMEM_SHARED((NP, 128), jnp.float32),
            pltpu.VMEM((EB, 128), jnp.int32),
            pltpu.VMEM((EB, 128), jnp.int32),
            pltpu.VMEM((EB, 128), jnp.int32),
            pltpu.VMEM((EB, 128), jnp.int32),
            pltpu.VMEM((128, 128), jnp.float32),
            pltpu.VMEM((128, 128), jnp.float32),
            pltpu.SemaphoreType.DMA,
            pltpu.SemaphoreType.DMA,
            pltpu.SemaphoreType.DMA,
        ],
    )(table, src2d, dst2d, zeros128)


def _tc_scale(xp, deg8):
    def body(x_ref, d_ref, t_ref, dis_ref):
        dis = lax.rsqrt(d_ref[...][:, 0:1] + 1.0)
        t = x_ref[...] * dis
        t_ref[0] = t
        t_ref[1] = t
        dis_ref[...] = dis

    return pl.pallas_call(
        body,
        grid=(NP // BR,),
        in_specs=[
            pl.BlockSpec((BR, DIN), lambda i: (i, 0)),
            pl.BlockSpec((BR, 16), lambda i: (i, 0)),
        ],
        out_specs=[
            pl.BlockSpec((2, BR, DIN), lambda i: (0, i, 0)),
            pl.BlockSpec((BR, 1), lambda i: (i, 0)),
        ],
        out_shape=[
            jax.ShapeDtypeStruct((2, NP, DIN), jnp.float32),
            jax.ShapeDtypeStruct((NP, 1), jnp.float32),
        ],
    )(xp, deg8)


def _tc_mid(aggx, t, dis2, W1, b1, gam, bet, mu, var, W2):
    def body(a_ref, t_ref, d_ref, w1_ref, b_ref, g_ref, be_ref, m_ref, v_ref,
             w2_ref, o_ref):
        sm = a_ref[0] + a_ref[1] + t_ref[0]
        u = jnp.dot(sm, w1_ref[...], preferred_element_type=jnp.float32)
        d = d_ref[...]
        g = d * u + b_ref[...]
        scale = g_ref[...] * lax.rsqrt(v_ref[...] + 1e-5)
        h = jnp.maximum(g * scale + (be_ref[...] - m_ref[...] * scale), 0.0)
        y2 = jnp.dot(h, w2_ref[...], preferred_element_type=jnp.float32) * d
        o_ref[0] = y2
        o_ref[1] = y2

    return pl.pallas_call(
        body,
        grid=(NP // BR,),
        in_specs=[
            pl.BlockSpec((2, BR, 128), lambda i: (0, i, 0)),
            pl.BlockSpec((2, BR, DIN), lambda i: (0, i, 0)),
            pl.BlockSpec((BR, 1), lambda i: (i, 0)),
            pl.BlockSpec((DIN, DH), lambda i: (0, 0)),
            pl.BlockSpec((1, DH), lambda i: (0, 0)),
            pl.BlockSpec((1, DH), lambda i: (0, 0)),
            pl.BlockSpec((1, DH), lambda i: (0, 0)),
            pl.BlockSpec((1, DH), lambda i: (0, 0)),
            pl.BlockSpec((1, DH), lambda i: (0, 0)),
            pl.BlockSpec((DH, DOUT), lambda i: (0, 0)),
        ],
        out_specs=pl.BlockSpec((2, BR, DOUT), lambda i: (0, i, 0)),
        out_shape=jax.ShapeDtypeStruct((2, NP, DOUT), jnp.float32),
    )(aggx, t, dis2, W1, b1, gam, bet, mu, var, W2)


def _tc_final(agg2, y2, dis2, b2):
    def body(a_ref, y_ref, d_ref, b_ref, o_ref):
        o_ref[...] = d_ref[...] * (a_ref[0] + a_ref[1] + y_ref[0]) + b_ref[...]

    return pl.pallas_call(
        body,
        grid=(NP // BR,),
        in_specs=[
            pl.BlockSpec((2, BR, DOUT), lambda i: (0, i, 0)),
            pl.BlockSpec((2, BR, DOUT), lambda i: (0, i, 0)),
            pl.BlockSpec((BR, 1), lambda i: (i, 0)),
            pl.BlockSpec((1, DOUT), lambda i: (0, 0)),
        ],
        out_specs=pl.BlockSpec((BR, DOUT), lambda i: (i, 0)),
        out_shape=jax.ShapeDtypeStruct((NP, DOUT), jnp.float32),
    )(agg2, y2, dis2, b2)


@jax.jit
def kernel(node_feat, edge_index, W1, b1, W2, b2, bn_gamma, bn_beta, bn_mean,
           bn_var):
    f32 = jnp.float32
    xp = jnp.zeros((NP, DIN), f32).at[:N].set(node_feat)
    pad = jnp.full((EP - E,), N, jnp.int32)
    src = jnp.concatenate([edge_index[0], pad])
    dst = jnp.concatenate([edge_index[1], pad])
    src2d = src.reshape(EPR, 128)
    dst2d = dst.reshape(EPR, 128)

    ones8 = jnp.ones((128, 16), f32)
    zeros8 = jnp.zeros((NP, 16), f32)
    zeros128 = jnp.zeros((NP, 128), f32)

    deg8 = _deg_kernel(dst2d, ones8, zeros8)
    t, dis2 = _tc_scale(xp, deg8)
    aggx = _agg_kernel(t, src2d, dst2d, zeros128)
    y2 = _tc_mid(aggx, t, dis2, W1, b1.reshape(1, DH), bn_gamma.reshape(1, DH),
                 bn_beta.reshape(1, DH), bn_mean.reshape(1, DH),
                 bn_var.reshape(1, DH), W2)
    agg2 = _agg_kernel(y2, src2d, dst2d, zeros128)
    out = _tc_final(agg2, y2, dis2, b2.reshape(1, DOUT))
    return out[:N]
